# baseline (speedup 1.0000x reference)
import functools

import jax
import jax.numpy as jnp
from jax import lax
from jax.experimental import pallas as pl
from jax.experimental.pallas import tpu as pltpu
from jax.experimental.pallas import tpu_sc as plsc

VOCAB = 1000000
DIM = 32
NUM_VALUES = 327680
BATCH = 16384

VB = 8192
NROWBLK = (VOCAB + 4 * VB - 1) // (4 * VB)
PACKED_ROWS = NROWBLK * VB
TABLE_ROWS = PACKED_ROWS * 4

NC = 2
NS = 16
NW = NC * NS
B_PER_W = NUM_VALUES // NW
CHUNK = 1280
NCH = B_PER_W // CHUNK
LPW = BATCH // NW
OFFW = LPW + 8


def _transpose_block(src_ref, dst_ref):
    src = src_ref[...]
    stacked = jnp.concatenate(
        [src[:, k * VB:(k + 1) * VB] for k in range(4)], axis=0
    )
    dst_ref[...] = stacked.T


def _pack_table(table_t):
    return pl.pallas_call(
        _transpose_block,
        grid=(NROWBLK,),
        in_specs=[pl.BlockSpec((DIM, 4 * VB), lambda g: (0, g))],
        out_specs=pl.BlockSpec((VB, 4 * DIM), lambda g: (g, 0)),
        out_shape=jax.ShapeDtypeStruct((PACKED_ROWS, 4 * DIM), jnp.float32),
    )(table_t)


_mesh = plsc.VectorSubcoreMesh(core_axis_name="c", subcore_axis_name="s")


@functools.partial(
    pl.kernel,
    mesh=_mesh,
    compiler_params=pltpu.CompilerParams(use_tc_tiling_on_sc=False),
    out_type=(
        jax.ShapeDtypeStruct((NUM_VALUES // 4, 4 * DIM), jnp.float32),
        jax.ShapeDtypeStruct((BATCH,), jnp.int32),
    ),
    scratch_types=[
        pltpu.VMEM((2, 4, CHUNK // 4), jnp.int32),
        pltpu.VMEM((2, 4, CHUNK // 4, DIM), jnp.float32),
        pltpu.VMEM((OFFW,), jnp.int32),
        pltpu.VMEM((LPW,), jnp.int32),
        pltpu.SemaphoreType.DMA,
        pltpu.SemaphoreType.DMA,
        pltpu.SemaphoreType.DMA,
        pltpu.SemaphoreType.DMA,
    ],
)
def _sc_lookup(values_h, offsets_h, table_h, emb_h, len_h,
               idx_v, rows_v, off_v, len_v, gsem0, gsem1, osem0, osem1):
    wid = lax.axis_index("s") * NC + lax.axis_index("c")
    gsem = (gsem0, gsem1)
    osem = (osem0, osem1)

    def remap(i, b, k):
        v = idx_v[b, k, pl.ds(i * 16, 16)]
        idx_v[b, k, pl.ds(i * 16, 16)] = (
            ((v >> 15) << 15) + ((v & 8191) << 2) + ((v >> 13) & 3)
        )
        return 0

    def fire(g, b):
        pltpu.sync_copy(values_h.at[wid, g], idx_v.at[b])
        for k in range(4):
            lax.fori_loop(0, CHUNK // 64, lambda i, c, k=k: remap(i, b, k), 0)
        return [
            pltpu.async_copy(
                table_h.at[idx_v.at[b, k]], rows_v.at[b, k], gsem[b]
            )
            for k in range(4)
        ]

    gh = [None, None]
    oh = [None, None]
    gh[0] = fire(0, 0)

    pltpu.sync_copy(offsets_h.at[pl.ds(wid * LPW, OFFW)], off_v)
    for j in range(LPW // 16):
        a = off_v[pl.ds(j * 16, 16)]
        b = off_v[pl.ds(j * 16 + 1, 16)]
        len_v[pl.ds(j * 16, 16)] = b - a
    pltpu.sync_copy(len_v, len_h.at[pl.ds(wid * LPW, LPW)])

    for g in range(NCH):
        cur = g & 1
        nxt = cur ^ 1
        if g + 1 < NCH:
            if oh[nxt] is not None:
                for h in oh[nxt]:
                    h.wait()
            gh[nxt] = fire(g + 1, nxt)
        for h in gh[cur]:
            h.wait()
        base4 = (wid * B_PER_W + g * CHUNK) // 4
        oh[cur] = [
            pltpu.async_copy(
                rows_v.at[cur, k],
                emb_h.at[pl.ds(base4, CHUNK // 4), pl.ds(k * DIM, DIM)],
                osem[cur],
            )
            for k in range(4)
        ]
    for hs in oh:
        for h in hs:
            h.wait()


def kernel(values, offsets, table):
    packed = _pack_table(table.T)
    table_rm = packed.reshape(TABLE_ROWS, DIM)
    values_r = jnp.swapaxes(
        values.astype(jnp.int32).reshape(NW, NCH, CHUNK // 4, 4), -1, -2
    )
    offsets_p = jnp.pad(offsets.astype(jnp.int32), (0, NW * LPW + OFFW - (BATCH + 1)))
    emb4, lens = _sc_lookup(values_r, offsets_p, table_rm)
    return emb4.reshape(NUM_VALUES, DIM), lens


# speedup vs baseline: 1.8983x; 1.8983x over previous
import functools

import jax
import jax.numpy as jnp
from jax import lax
from jax.experimental import pallas as pl
from jax.experimental.pallas import tpu as pltpu
from jax.experimental.pallas import tpu_sc as plsc

VOCAB = 1000000
DIM = 32
NUM_VALUES = 327680
BATCH = 16384

VB = 16384
NROWBLK = (VOCAB + 4 * VB - 1) // (4 * VB)
PACKED_ROWS = NROWBLK * VB
TABLE_ROWS = PACKED_ROWS * 4

NC = 2
NS = 16
NW = NC * NS
B_PER_W = NUM_VALUES // NW
CHUNK = 1280
NCH = B_PER_W // CHUNK
LPW = BATCH // NW
OFFW = LPW + 8


def _transpose_block(src_ref, dst_ref):
    src = src_ref[...]
    stacked = jnp.concatenate(
        [src[:, k * VB:(k + 1) * VB] for k in range(4)], axis=0
    )
    dst_ref[...] = stacked.T


def _pack_table(table_t):
    return pl.pallas_call(
        _transpose_block,
        grid=(NROWBLK,),
        in_specs=[pl.BlockSpec((DIM, 4 * VB), lambda g: (0, g))],
        out_specs=pl.BlockSpec((VB, 4 * DIM), lambda g: (g, 0)),
        out_shape=jax.ShapeDtypeStruct((PACKED_ROWS, 4 * DIM), jnp.float32),
    )(table_t)


_mesh = plsc.VectorSubcoreMesh(core_axis_name="c", subcore_axis_name="s")


@functools.partial(
    pl.kernel,
    mesh=_mesh,
    compiler_params=pltpu.CompilerParams(use_tc_tiling_on_sc=False),
    out_type=(
        jax.ShapeDtypeStruct((NUM_VALUES, 128), jnp.float32),
        jax.ShapeDtypeStruct((NW, LPW), jnp.int32),
    ),
    scratch_types=[
        pltpu.VMEM((2, CHUNK), jnp.int32),
        pltpu.VMEM((2, CHUNK, DIM), jnp.float32),
        pltpu.VMEM((OFFW,), jnp.int32),
        pltpu.VMEM((LPW,), jnp.int32),
        pltpu.SemaphoreType.DMA,
        pltpu.SemaphoreType.DMA,
        pltpu.SemaphoreType.DMA,
        pltpu.SemaphoreType.DMA,
    ],
)
def _sc_lookup(values_h, offsets_h, table_h, emb_h, len_h,
               idx_v, rows_v, off_v, len_v, gsem0, gsem1, osem0, osem1):
    wid = lax.axis_index("s") * NC + lax.axis_index("c")
    gsem = (gsem0, gsem1)
    osem = (osem0, osem1)

    def remap(i, b):
        v = idx_v[b, pl.ds(i * 16, 16)]
        idx_v[b, pl.ds(i * 16, 16)] = (
            ((v >> 16) << 16) + ((v & 16383) << 2) + ((v >> 14) & 3)
        )
        return 0

    def fire(g, b):
        pltpu.sync_copy(values_h.at[wid, g], idx_v.at[b])
        lax.fori_loop(0, CHUNK // 16, lambda i, c: remap(i, b), 0)
        return pltpu.async_copy(table_h.at[idx_v.at[b]], rows_v.at[b], gsem[b])

    gh = [None, None]
    oh = [None, None]
    gh[0] = fire(0, 0)

    pltpu.sync_copy(offsets_h.at[pl.ds(wid * LPW, OFFW)], off_v)
    for j in range(LPW // 16):
        a = off_v[pl.ds(j * 16, 16)]
        b = off_v[pl.ds(j * 16 + 1, 16)]
        len_v[pl.ds(j * 16, 16)] = b - a
    pltpu.sync_copy(len_v, len_h.at[wid])

    for g in range(NCH):
        cur = g & 1
        nxt = cur ^ 1
        if g + 1 < NCH:
            if oh[nxt] is not None:
                oh[nxt].wait()
            gh[nxt] = fire(g + 1, nxt)
        gh[cur].wait()
        oh[cur] = pltpu.async_copy(
            rows_v.at[cur],
            emb_h.at[pl.ds(wid * B_PER_W + g * CHUNK, CHUNK), pl.ds(0, DIM)],
            osem[cur],
        )
    oh[0].wait()
    oh[1].wait()


def kernel(values, offsets, table):
    packed = _pack_table(table.T)
    table_rm = packed.reshape(TABLE_ROWS, DIM)
    values_r = values.astype(jnp.int32).reshape(NW, NCH, CHUNK)
    offsets_p = jnp.pad(offsets.astype(jnp.int32), (0, NW * LPW + OFFW - (BATCH + 1)))
    emb_pad, lens = _sc_lookup(values_r, offsets_p, table_rm)
    return emb_pad[:, :DIM], lens.reshape(BATCH)
